# baseline (speedup 1.0000x reference)
import functools

import jax
import jax.numpy as jnp
import numpy as np
from jax import lax
from jax.experimental import pallas as pl
from jax.experimental.pallas import tpu as pltpu
from jax.experimental.pallas import tpu_sc as plsc

_N = 10000
_E = 640000
_HEADS = 8
_HDIM = 8

_NC = 2
_NS = 16
_NW = _NC * _NS
_CH = 128
_GRP = 4
_GE = _GRP * _CH
_EPW = 20480
_NGRP = _EPW // _GE
_EPAD = _NW * _EPW
_NPAD = 10112
_RPT = _NPAD // _NS
_WL1 = 80
_WL2 = 16

_f32 = jnp.float32


def _prep1_body(x_ref, w1_ref, ms_ref, md_ref, e8_ref, g1_ref, g2_ref, p16_ref,
                p1_ref, d1_ref, init_ref):
    xp = jnp.dot(x_ref[:], w1_ref[:], preferred_element_type=_f32, precision=jax.lax.Precision.HIGHEST)
    a_s = jnp.dot(xp, ms_ref[:], preferred_element_type=_f32, precision=jax.lax.Precision.HIGHEST)
    a_d = jnp.dot(xp, md_ref[:], preferred_element_type=_f32, precision=jax.lax.Precision.HIGHEST)
    al = a_s + a_d
    f_self = jnp.exp(jnp.maximum(al, 0.2 * al))
    f_exp = jnp.dot(f_self, e8_ref[:], preferred_element_type=_f32, precision=jax.lax.Precision.HIGHEST)
    p1_ref[:] = (jnp.dot(xp, g1_ref[:], preferred_element_type=_f32, precision=jax.lax.Precision.HIGHEST)
                 + jnp.dot(a_s, g2_ref[:], preferred_element_type=_f32, precision=jax.lax.Precision.HIGHEST))
    d1_ref[:] = jnp.dot(a_d, p16_ref[:], preferred_element_type=_f32, precision=jax.lax.Precision.HIGHEST)
    init_ref[:] = (jnp.dot(xp * f_exp, g1_ref[:], preferred_element_type=_f32, precision=jax.lax.Precision.HIGHEST)
                   + jnp.dot(f_self, g2_ref[:], preferred_element_type=_f32, precision=jax.lax.Precision.HIGHEST))


def _mid_body(p0_ref, p1_ref, i1_ref, b1_ref, w2_ref, as2_ref, ad2_ref,
              hn_ref, hd_ref, k1_ref, k2_ref, k3_ref, one8_ref,
              t2_ref, d2_ref, init2_ref):
    s = p0_ref[:] + p1_ref[:] + i1_ref[:]
    num = jnp.dot(s, hn_ref[:], preferred_element_type=_f32, precision=jax.lax.Precision.HIGHEST)
    den = jnp.dot(s, hd_ref[:], preferred_element_type=_f32, precision=jax.lax.Precision.HIGHEST)
    h = jnp.maximum(num / den + b1_ref[:], 0.0)
    hp = jnp.dot(h, w2_ref[:], preferred_element_type=_f32, precision=jax.lax.Precision.HIGHEST)
    a2s = jnp.dot(hp, as2_ref[:], preferred_element_type=_f32, precision=jax.lax.Precision.HIGHEST)
    a2d = jnp.dot(hp, ad2_ref[:], preferred_element_type=_f32, precision=jax.lax.Precision.HIGHEST)
    al = a2s + a2d
    f2 = jnp.exp(jnp.maximum(al, 0.2 * al))
    f2e = jnp.dot(f2, one8_ref[:], preferred_element_type=_f32, precision=jax.lax.Precision.HIGHEST)
    t2_ref[:] = (jnp.dot(hp, k1_ref[:], preferred_element_type=_f32, precision=jax.lax.Precision.HIGHEST)
                 + jnp.dot(a2s, k2_ref[:], preferred_element_type=_f32, precision=jax.lax.Precision.HIGHEST))
    d2_ref[:] = jnp.dot(a2d, k3_ref[:], preferred_element_type=_f32, precision=jax.lax.Precision.HIGHEST)
    init2_ref[:] = (jnp.dot(hp * f2e, k1_ref[:], preferred_element_type=_f32, precision=jax.lax.Precision.HIGHEST)
                    + jnp.dot(f2, k2_ref[:], preferred_element_type=_f32, precision=jax.lax.Precision.HIGHEST))


def _fin_body(q0_ref, q1_ref, i2_ref, b2_ref, fw_ref, fb_ref, qn_ref, qd_ref,
              emb_ref, sc_ref):
    s = q0_ref[:] + q1_ref[:] + i2_ref[:]
    num = jnp.dot(s, qn_ref[:], preferred_element_type=_f32, precision=jax.lax.Precision.HIGHEST)
    den = jnp.dot(s, qd_ref[:], preferred_element_type=_f32, precision=jax.lax.Precision.HIGHEST)
    emb = num / den + b2_ref[:]
    emb_ref[:] = emb
    sc_ref[:] = jnp.dot(emb, fw_ref[:], preferred_element_type=_f32, precision=jax.lax.Precision.HIGHEST) + fb_ref[:]


def _make_edge_kernel(width):
    mesh = plsc.VectorSubcoreMesh(core_axis_name="c", subcore_axis_name="s",
                                  num_cores=_NC, num_subcores=_NS)
    nvec = width // 16

    def body(tbl, dtbl, eis, eid, out, acc, isrc, idst, rows, drows, sem):
        cid = lax.axis_index("c")
        sid = lax.axis_index("s")
        wid = sid * _NC + cid
        lane = lax.iota(jnp.int32, 16)
        zero16 = jnp.zeros((16,), _f32)

        def zrow(r, carry):
            for v in range(nvec):
                rows[r, pl.ds(16 * v, 16)] = zero16
            return carry
        lax.fori_loop(0, _GE, zrow, None)
        off = 0
        while off < _RPT:
            n = min(_GE, _RPT - off)
            pltpu.sync_copy(rows.at[pl.ds(0, n)],
                            acc.at[pl.ds(sid * _RPT + off, n)])
            off += n
        plsc.subcore_barrier()

        if width == _WL1:
            half = jnp.where(lane < 8, 0, 1).astype(jnp.int32)
            cidx = [64 + 2 * j + half for j in range(4)]

            def edge(e):
                av = rows[e, pl.ds(64, 16)]
                dv = drows[e, pl.ds(0, 16)]
                al = av + dv
                f = jnp.exp(jnp.maximum(al, 0.2 * al))
                rows[e, pl.ds(64, 16)] = f
                es = jnp.full((16,), e, jnp.int32)
                for j in range(4):
                    fe = plsc.load_gather(rows, [es, cidx[j]])
                    rows[e, pl.ds(16 * j, 16)] = fe * rows[e, pl.ds(16 * j, 16)]
        else:
            m8 = jnp.where(lane < 8, 1.0, 0.0).astype(_f32)
            e8 = jnp.where(lane == 8, 1.0, 0.0).astype(_f32)
            i8 = jnp.full((16,), 8, jnp.int32)
            i0 = jnp.zeros((16,), jnp.int32)

            def edge(e):
                es = jnp.full((16,), e, jnp.int32)
                av = rows[e, pl.ds(0, 16)]
                a2s = plsc.load_gather(rows, [es, i8])
                a2d = plsc.load_gather(drows, [es, i0])
                al = a2s + a2d
                f = jnp.exp(jnp.maximum(al, 0.2 * al))
                rows[e, pl.ds(0, 16)] = f * (av * m8 + e8)

        def group(g, carry):
            pltpu.sync_copy(eis.at[wid, g], isrc)
            pltpu.sync_copy(eid.at[wid, g], idst)
            cps = []
            for j in range(_GRP):
                cps.append(pltpu.async_copy(
                    tbl.at[isrc.at[j]], rows.at[pl.ds(j * _CH, _CH)], sem))
                cps.append(pltpu.async_copy(
                    dtbl.at[idst.at[j]], drows.at[pl.ds(j * _CH, _CH)], sem))
            for c in cps:
                c.wait()
            plsc.parallel_loop(0, _GE, 1, unroll=16)(edge)
            for j in range(_GRP):
                pltpu.sync_copy(rows.at[pl.ds(j * _CH, _CH)],
                                acc.at[idst.at[j]], add=True)
            return carry
        lax.fori_loop(0, _NGRP, group, None)

        plsc.subcore_barrier()
        pltpu.sync_copy(acc.at[pl.ds(sid * _RPT, _RPT)],
                        out.at[cid, pl.ds(sid * _RPT, _RPT)])

    return pl.kernel(
        body,
        out_type=jax.ShapeDtypeStruct((_NC, _NPAD, width), _f32),
        mesh=mesh,
        compiler_params=pltpu.CompilerParams(use_tc_tiling_on_sc=False,
                                             needs_layout_passes=False),
        scratch_types=[
            pltpu.VMEM_SHARED((_NPAD, width), _f32),
            pltpu.VMEM((_GRP, _CH), jnp.int32),
            pltpu.VMEM((_GRP, _CH), jnp.int32),
            pltpu.VMEM((_GE, width), _f32),
            pltpu.VMEM((_GE, 16), _f32),
            pltpu.SemaphoreType.DMA,
        ],
    )


_edge_kernel = functools.cache(_make_edge_kernel)


def _static_mats():
    g1 = np.zeros((64, _WL1), np.float32)
    g1[:64, :64] = np.eye(64)
    g2 = np.zeros((8, _WL1), np.float32)
    g2[np.arange(8), 64 + np.arange(8)] = 1.0
    e8 = np.zeros((8, 64), np.float32)
    e8[np.arange(64) // 8, np.arange(64)] = 1.0
    p16 = np.zeros((8, 16), np.float32)
    p16[np.arange(8), np.arange(8)] = 1.0
    hn = np.zeros((_WL1, 64), np.float32)
    hn[:64, :64] = np.eye(64)
    hd = np.zeros((_WL1, 64), np.float32)
    hd[64 + np.arange(64) // 8, np.arange(64)] = 1.0
    k1 = np.zeros((8, _WL2), np.float32)
    k1[np.arange(8), np.arange(8)] = 1.0
    k2 = np.zeros((1, _WL2), np.float32)
    k2[0, 8] = 1.0
    k3 = np.zeros((1, _WL2), np.float32)
    k3[0, 0] = 1.0
    qn = np.zeros((_WL2, 8), np.float32)
    qn[:8, :8] = np.eye(8)
    qd = np.zeros((_WL2, 8), np.float32)
    qd[8, :] = 1.0
    one8 = np.ones((1, 8), np.float32)
    return dict(g1=g1, g2=g2, e8=e8, p16=p16, hn=hn, hd=hd,
                k1=k1, k2=k2, k3=k3, qn=qn, qd=qd, one8=one8)


_MATS = _static_mats()


def kernel(x, edge_index, W1, att_src1, att_dst1, b1, W2, att_src2, att_dst2,
           b2, fc3_w, fc3_b):
    m = _MATS
    ei = edge_index.astype(jnp.int32)
    npe = _EPAD - _E
    ar = jnp.arange(npe, dtype=jnp.int32)
    src = jnp.concatenate([ei[0], ar % _N])
    dst = jnp.concatenate([ei[1], _N + ar % (_NPAD - _N)])
    eis = src.reshape(_NW, _NGRP, _GRP, _CH)
    eid = dst.reshape(_NW, _NGRP, _GRP, _CH)

    ar64 = jnp.arange(64)
    ms = jnp.zeros((64, 8), _f32).at[ar64, ar64 // 8].set(att_src1.reshape(-1))
    md = jnp.zeros((64, 8), _f32).at[ar64, ar64 // 8].set(att_dst1.reshape(-1))

    blk = 2000
    full = lambda s: pl.BlockSpec(s, lambda i: (0, 0))
    rows = lambda c: pl.BlockSpec((blk, c), lambda i: (i, 0))
    p1t, d1t, init1 = pl.pallas_call(
        _prep1_body,
        grid=(_N // blk,),
        in_specs=[rows(128), full((128, 64)), full((64, 8)), full((64, 8)),
                  full((8, 64)), full((64, _WL1)), full((8, _WL1)),
                  full((8, 16))],
        out_specs=[rows(_WL1), rows(16), rows(_WL1)],
        out_shape=[jax.ShapeDtypeStruct((_N, _WL1), _f32),
                   jax.ShapeDtypeStruct((_N, 16), _f32),
                   jax.ShapeDtypeStruct((_N, _WL1), _f32)],
    )(x, W1, ms, md, m["e8"], m["g1"], m["g2"], m["p16"])

    d1p = jnp.zeros((_NPAD, 16), _f32).at[:_N].set(d1t)
    i1p = jnp.zeros((_NPAD, _WL1), _f32).at[:_N].set(init1)

    parts1 = _edge_kernel(_WL1)(p1t, d1p, eis, eid)

    blk2 = 1264
    rows2 = lambda c: pl.BlockSpec((blk2, c), lambda i: (i, 0))
    t2, d2, init2 = pl.pallas_call(
        _mid_body,
        grid=(_NPAD // blk2,),
        in_specs=[rows2(_WL1), rows2(_WL1), rows2(_WL1), full((1, 64)),
                  full((64, 8)), full((8, 1)), full((8, 1)),
                  full((_WL1, 64)), full((_WL1, 64)), full((8, _WL2)),
                  full((1, _WL2)), full((1, _WL2)), full((1, 8))],
        out_specs=[rows2(_WL2), rows2(_WL2), rows2(_WL2)],
        out_shape=[jax.ShapeDtypeStruct((_NPAD, _WL2), _f32),
                   jax.ShapeDtypeStruct((_NPAD, _WL2), _f32),
                   jax.ShapeDtypeStruct((_NPAD, _WL2), _f32)],
    )(parts1[0], parts1[1], i1p, b1.reshape(1, 64), W2,
      att_src2.reshape(8, 1), att_dst2.reshape(8, 1),
      m["hn"], m["hd"], m["k1"], m["k2"], m["k3"], m["one8"])

    parts2 = _edge_kernel(_WL2)(t2, d2, eis, eid)

    embp, scorep = pl.pallas_call(
        _fin_body,
        grid=(_NPAD // blk2,),
        in_specs=[rows2(_WL2), rows2(_WL2), rows2(_WL2), full((1, 8)),
                  full((8, 1)), full((1, 1)), full((_WL2, 8)),
                  full((_WL2, 8))],
        out_specs=[rows2(8), rows2(1)],
        out_shape=[jax.ShapeDtypeStruct((_NPAD, 8), _f32),
                   jax.ShapeDtypeStruct((_NPAD, 1), _f32)],
    )(parts2[0], parts2[1], init2, b2.reshape(1, 8), fc3_w,
      fc3_b.reshape(1, 1), m["qn"], m["qd"])

    return (embp[:_N], scorep[:_N])


# speedup vs baseline: 1.2428x; 1.2428x over previous
import functools

import jax
import jax.numpy as jnp
import numpy as np
from jax import lax
from jax.experimental import pallas as pl
from jax.experimental.pallas import tpu as pltpu
from jax.experimental.pallas import tpu_sc as plsc

_N = 10000
_E = 640000
_HEADS = 8
_HDIM = 8

_NC = 2
_NS = 16
_NW = _NC * _NS
_CH = 128
_GRP = 4
_GE = _GRP * _CH
_EPW = 20480
_NGRP = _EPW // _GE
_EPAD = _NW * _EPW
_NPAD = 10112
_RPT = _NPAD // _NS
_WL1 = 80
_WL2 = 16

_f32 = jnp.float32


def _prep1_body(x_ref, w1_ref, ms_ref, md_ref, e8_ref, g1_ref, g2_ref, p16_ref,
                p1_ref, d1_ref, init_ref):
    xp = jnp.dot(x_ref[:], w1_ref[:], preferred_element_type=_f32)
    a_s = jnp.dot(xp, ms_ref[:], preferred_element_type=_f32, precision=jax.lax.Precision.HIGHEST)
    a_d = jnp.dot(xp, md_ref[:], preferred_element_type=_f32, precision=jax.lax.Precision.HIGHEST)
    al = a_s + a_d
    f_self = jnp.exp(jnp.maximum(al, 0.2 * al))
    f_exp = jnp.dot(f_self, e8_ref[:], preferred_element_type=_f32, precision=jax.lax.Precision.HIGHEST)
    p1_ref[:] = (jnp.dot(xp, g1_ref[:], preferred_element_type=_f32, precision=jax.lax.Precision.HIGHEST)
                 + jnp.dot(a_s, g2_ref[:], preferred_element_type=_f32, precision=jax.lax.Precision.HIGHEST))
    d1_ref[:] = jnp.dot(a_d, p16_ref[:], preferred_element_type=_f32, precision=jax.lax.Precision.HIGHEST)
    init_ref[:] = (jnp.dot(xp * f_exp, g1_ref[:], preferred_element_type=_f32, precision=jax.lax.Precision.HIGHEST)
                   + jnp.dot(f_self, g2_ref[:], preferred_element_type=_f32, precision=jax.lax.Precision.HIGHEST))


def _mid_body(p0_ref, p1_ref, i1_ref, b1_ref, w2_ref, as2_ref, ad2_ref,
              hn_ref, hd_ref, k1_ref, k2_ref, k3_ref, one8_ref,
              t2_ref, d2_ref, init2_ref):
    s = p0_ref[:] + p1_ref[:] + i1_ref[:]
    num = jnp.dot(s, hn_ref[:], preferred_element_type=_f32, precision=jax.lax.Precision.HIGHEST)
    den = jnp.dot(s, hd_ref[:], preferred_element_type=_f32, precision=jax.lax.Precision.HIGHEST)
    h = jnp.maximum(num / den + b1_ref[:], 0.0)
    hp = jnp.dot(h, w2_ref[:], preferred_element_type=_f32)
    a2s = jnp.dot(hp, as2_ref[:], preferred_element_type=_f32, precision=jax.lax.Precision.HIGHEST)
    a2d = jnp.dot(hp, ad2_ref[:], preferred_element_type=_f32, precision=jax.lax.Precision.HIGHEST)
    al = a2s + a2d
    f2 = jnp.exp(jnp.maximum(al, 0.2 * al))
    f2e = jnp.dot(f2, one8_ref[:], preferred_element_type=_f32, precision=jax.lax.Precision.HIGHEST)
    t2_ref[:] = (jnp.dot(hp, k1_ref[:], preferred_element_type=_f32, precision=jax.lax.Precision.HIGHEST)
                 + jnp.dot(a2s, k2_ref[:], preferred_element_type=_f32, precision=jax.lax.Precision.HIGHEST))
    d2_ref[:] = jnp.dot(a2d, k3_ref[:], preferred_element_type=_f32, precision=jax.lax.Precision.HIGHEST)
    init2_ref[:] = (jnp.dot(hp * f2e, k1_ref[:], preferred_element_type=_f32, precision=jax.lax.Precision.HIGHEST)
                    + jnp.dot(f2, k2_ref[:], preferred_element_type=_f32, precision=jax.lax.Precision.HIGHEST))


def _fin_body(q0_ref, q1_ref, i2_ref, b2_ref, fw_ref, fb_ref, qn_ref, qd_ref,
              emb_ref, sc_ref):
    s = q0_ref[:] + q1_ref[:] + i2_ref[:]
    num = jnp.dot(s, qn_ref[:], preferred_element_type=_f32, precision=jax.lax.Precision.HIGHEST)
    den = jnp.dot(s, qd_ref[:], preferred_element_type=_f32, precision=jax.lax.Precision.HIGHEST)
    emb = num / den + b2_ref[:]
    emb_ref[:] = emb
    sc_ref[:] = jnp.dot(emb, fw_ref[:], preferred_element_type=_f32) + fb_ref[:]


def _make_edge_kernel(width):
    grp = 2 if width == _WL1 else 4
    ge = grp * _CH
    ng = _EPW // ge
    mesh = plsc.VectorSubcoreMesh(core_axis_name="c", subcore_axis_name="s",
                                  num_cores=_NC, num_subcores=_NS)
    nvec = width // 16

    def body(tbl, dtbl, eis, eid, out, acc,
             isrc0, idst0, rows0, drows0, isrc1, idst1, rows1, drows1,
             sem0, sem1):
        cid = lax.axis_index("c")
        sid = lax.axis_index("s")
        wid = sid * _NC + cid
        lane = lax.iota(jnp.int32, 16)
        zero16 = jnp.zeros((16,), _f32)
        bufs = [(isrc0, idst0, rows0, drows0, sem0),
                (isrc1, idst1, rows1, drows1, sem1)]

        def zrow(r, carry):
            for v in range(nvec):
                rows0[r, pl.ds(16 * v, 16)] = zero16
            return carry
        lax.fori_loop(0, ge, zrow, None)
        off = 0
        while off < _RPT:
            n = min(ge, _RPT - off)
            pltpu.sync_copy(rows0.at[pl.ds(0, n)],
                            acc.at[pl.ds(sid * _RPT + off, n)])
            off += n
        plsc.subcore_barrier()

        if width == _WL1:
            half = jnp.where(lane < 8, 0, 1).astype(jnp.int32)
            cidx = [64 + 2 * j + half for j in range(4)]

            def make_edge(rows_, drows_):
                def edge(e):
                    av = rows_[e, pl.ds(64, 16)]
                    dv = drows_[e, pl.ds(0, 16)]
                    al = av + dv
                    f = jnp.exp(jnp.maximum(al, 0.2 * al))
                    rows_[e, pl.ds(64, 16)] = f
                    es = jnp.full((16,), e, jnp.int32)
                    for j in range(4):
                        fe = plsc.load_gather(rows_, [es, cidx[j]])
                        rows_[e, pl.ds(16 * j, 16)] = fe * rows_[e, pl.ds(16 * j, 16)]
                return edge
        else:
            m8 = jnp.where(lane < 8, 1.0, 0.0).astype(_f32)
            e8 = jnp.where(lane == 8, 1.0, 0.0).astype(_f32)
            i8 = jnp.full((16,), 8, jnp.int32)
            i0 = jnp.zeros((16,), jnp.int32)

            def make_edge(rows_, drows_):
                def edge(e):
                    es = jnp.full((16,), e, jnp.int32)
                    av = rows_[e, pl.ds(0, 16)]
                    a2s = plsc.load_gather(rows_, [es, i8])
                    a2d = plsc.load_gather(drows_, [es, i0])
                    al = a2s + a2d
                    f = jnp.exp(jnp.maximum(al, 0.2 * al))
                    rows_[e, pl.ds(0, 16)] = f * (av * m8 + e8)
                return edge

        def fire(p, g):
            isrc_, idst_, rows_, drows_, sem_ = bufs[p]
            pltpu.sync_copy(eis.at[wid, pl.ds(g * grp, grp)], isrc_)
            pltpu.sync_copy(eid.at[wid, pl.ds(g * grp, grp)], idst_)
            for j in range(grp):
                pltpu.async_copy(tbl.at[isrc_.at[j]],
                                 rows_.at[pl.ds(j * _CH, _CH)], sem_)
                pltpu.async_copy(dtbl.at[idst_.at[j]],
                                 drows_.at[pl.ds(j * _CH, _CH)], sem_)

        def drain(p):
            isrc_, idst_, rows_, drows_, sem_ = bufs[p]
            for j in range(grp):
                pltpu.make_async_copy(tbl.at[isrc_.at[j]],
                                      rows_.at[pl.ds(j * _CH, _CH)], sem_).wait()
                pltpu.make_async_copy(dtbl.at[idst_.at[j]],
                                      drows_.at[pl.ds(j * _CH, _CH)], sem_).wait()

        def work(p):
            isrc_, idst_, rows_, drows_, sem_ = bufs[p]
            plsc.parallel_loop(0, ge, 1, unroll=16)(make_edge(rows_, drows_))
            for j in range(grp):
                pltpu.sync_copy(rows_.at[pl.ds(j * _CH, _CH)],
                                acc.at[idst_.at[j]], add=True)

        fire(0, 0)

        def pair(k, carry):
            g0 = 2 * k
            drain(0)
            fire(1, g0 + 1)
            work(0)
            drain(1)
            @pl.when(k < ng // 2 - 1)
            def _():
                fire(0, g0 + 2)
            work(1)
            return carry
        lax.fori_loop(0, ng // 2, pair, None)

        plsc.subcore_barrier()
        pltpu.sync_copy(acc.at[pl.ds(sid * _RPT, _RPT)],
                        out.at[cid, pl.ds(sid * _RPT, _RPT)])

    return pl.kernel(
        body,
        out_type=jax.ShapeDtypeStruct((_NC, _NPAD, width), _f32),
        mesh=mesh,
        compiler_params=pltpu.CompilerParams(use_tc_tiling_on_sc=False,
                                             needs_layout_passes=False),
        scratch_types=[
            pltpu.VMEM_SHARED((_NPAD, width), _f32),
            pltpu.VMEM((grp, _CH), jnp.int32),
            pltpu.VMEM((grp, _CH), jnp.int32),
            pltpu.VMEM((ge, width), _f32),
            pltpu.VMEM((ge, 16), _f32),
            pltpu.VMEM((grp, _CH), jnp.int32),
            pltpu.VMEM((grp, _CH), jnp.int32),
            pltpu.VMEM((ge, width), _f32),
            pltpu.VMEM((ge, 16), _f32),
            pltpu.SemaphoreType.DMA,
            pltpu.SemaphoreType.DMA,
        ],
    )


_edge_kernel = functools.cache(_make_edge_kernel)


def _static_mats():
    g1 = np.zeros((64, _WL1), np.float32)
    g1[:64, :64] = np.eye(64)
    g2 = np.zeros((8, _WL1), np.float32)
    g2[np.arange(8), 64 + np.arange(8)] = 1.0
    e8 = np.zeros((8, 64), np.float32)
    e8[np.arange(64) // 8, np.arange(64)] = 1.0
    p16 = np.zeros((8, 16), np.float32)
    p16[np.arange(8), np.arange(8)] = 1.0
    hn = np.zeros((_WL1, 64), np.float32)
    hn[:64, :64] = np.eye(64)
    hd = np.zeros((_WL1, 64), np.float32)
    hd[64 + np.arange(64) // 8, np.arange(64)] = 1.0
    k1 = np.zeros((8, _WL2), np.float32)
    k1[np.arange(8), np.arange(8)] = 1.0
    k2 = np.zeros((1, _WL2), np.float32)
    k2[0, 8] = 1.0
    k3 = np.zeros((1, _WL2), np.float32)
    k3[0, 0] = 1.0
    qn = np.zeros((_WL2, 8), np.float32)
    qn[:8, :8] = np.eye(8)
    qd = np.zeros((_WL2, 8), np.float32)
    qd[8, :] = 1.0
    one8 = np.ones((1, 8), np.float32)
    return dict(g1=g1, g2=g2, e8=e8, p16=p16, hn=hn, hd=hd,
                k1=k1, k2=k2, k3=k3, qn=qn, qd=qd, one8=one8)


_MATS = _static_mats()


def kernel(x, edge_index, W1, att_src1, att_dst1, b1, W2, att_src2, att_dst2,
           b2, fc3_w, fc3_b):
    m = _MATS
    ei = edge_index.astype(jnp.int32)
    npe = _EPAD - _E
    ar = jnp.arange(npe, dtype=jnp.int32)
    esrc = jnp.concatenate([ei[0], ar % _N])
    edst = jnp.concatenate([ei[1], _N + ar % (_NPAD - _N)])
    eis = esrc.reshape(_NW, _EPW // _CH, _CH)
    eid = edst.reshape(_NW, _EPW // _CH, _CH)

    ar64 = jnp.arange(64)
    ms = jnp.zeros((64, 8), _f32).at[ar64, ar64 // 8].set(att_src1.reshape(-1))
    md = jnp.zeros((64, 8), _f32).at[ar64, ar64 // 8].set(att_dst1.reshape(-1))

    blk = 2000
    full = lambda s: pl.BlockSpec(s, lambda i: (0, 0))
    rows = lambda c: pl.BlockSpec((blk, c), lambda i: (i, 0))
    p1t, d1t, init1 = pl.pallas_call(
        _prep1_body,
        grid=(_N // blk,),
        in_specs=[rows(128), full((128, 64)), full((64, 8)), full((64, 8)),
                  full((8, 64)), full((64, _WL1)), full((8, _WL1)),
                  full((8, 16))],
        out_specs=[rows(_WL1), rows(16), rows(_WL1)],
        out_shape=[jax.ShapeDtypeStruct((_N, _WL1), _f32),
                   jax.ShapeDtypeStruct((_N, 16), _f32),
                   jax.ShapeDtypeStruct((_N, _WL1), _f32)],
    )(x, W1, ms, md, m["e8"], m["g1"], m["g2"], m["p16"])

    d1p = jnp.zeros((_NPAD, 16), _f32).at[:_N].set(d1t)
    i1p = jnp.zeros((_NPAD, _WL1), _f32).at[:_N].set(init1)

    parts1 = _edge_kernel(_WL1)(p1t, d1p, eis, eid)

    blk2 = 1264
    rows2 = lambda c: pl.BlockSpec((blk2, c), lambda i: (i, 0))
    t2, d2, init2 = pl.pallas_call(
        _mid_body,
        grid=(_NPAD // blk2,),
        in_specs=[rows2(_WL1), rows2(_WL1), rows2(_WL1), full((1, 64)),
                  full((64, 8)), full((8, 1)), full((8, 1)),
                  full((_WL1, 64)), full((_WL1, 64)), full((8, _WL2)),
                  full((1, _WL2)), full((1, _WL2)), full((1, 8))],
        out_specs=[rows2(_WL2), rows2(_WL2), rows2(_WL2)],
        out_shape=[jax.ShapeDtypeStruct((_NPAD, _WL2), _f32),
                   jax.ShapeDtypeStruct((_NPAD, _WL2), _f32),
                   jax.ShapeDtypeStruct((_NPAD, _WL2), _f32)],
    )(parts1[0], parts1[1], i1p, b1.reshape(1, 64), W2,
      att_src2.reshape(8, 1), att_dst2.reshape(8, 1),
      m["hn"], m["hd"], m["k1"], m["k2"], m["k3"], m["one8"])

    parts2 = _edge_kernel(_WL2)(t2, d2, eis, eid)

    embp, scorep = pl.pallas_call(
        _fin_body,
        grid=(_NPAD // blk2,),
        in_specs=[rows2(_WL2), rows2(_WL2), rows2(_WL2), full((1, 8)),
                  full((8, 1)), full((1, 1)), full((_WL2, 8)),
                  full((_WL2, 8))],
        out_specs=[rows2(8), rows2(1)],
        out_shape=[jax.ShapeDtypeStruct((_NPAD, 8), _f32),
                   jax.ShapeDtypeStruct((_NPAD, 1), _f32)],
    )(parts2[0], parts2[1], init2, b2.reshape(1, 8), fc3_w,
      fc3_b.reshape(1, 1), m["qn"], m["qd"])

    return (embp[:_N], scorep[:_N])


# speedup vs baseline: 1.2548x; 1.0097x over previous
import functools

import jax
import jax.numpy as jnp
import numpy as np
from jax import lax
from jax.experimental import pallas as pl
from jax.experimental.pallas import tpu as pltpu
from jax.experimental.pallas import tpu_sc as plsc

_N = 10000
_E = 640000
_HEADS = 8
_HDIM = 8

_NC = 2
_NS = 16
_NW = _NC * _NS
_CH = 128
_GRP = 4
_GE = _GRP * _CH
_EPW = 20480
_NGRP = _EPW // _GE
_EPAD = _NW * _EPW
_NPAD = 10112
_RPT = _NPAD // _NS
_WL1 = 80
_WL2 = 16

_f32 = jnp.float32


def _prep1_body(x_ref, w1_ref, ms_ref, md_ref, e8_ref, g1_ref, g2_ref, p16_ref,
                p1_ref, d1_ref, init_ref):
    xp = jnp.dot(x_ref[:], w1_ref[:], preferred_element_type=_f32)
    a_s = jnp.dot(xp, ms_ref[:], preferred_element_type=_f32, precision=jax.lax.Precision.HIGHEST)
    a_d = jnp.dot(xp, md_ref[:], preferred_element_type=_f32, precision=jax.lax.Precision.HIGHEST)
    al = a_s + a_d
    f_self = jnp.exp(jnp.maximum(al, 0.2 * al))
    f_exp = jnp.dot(f_self, e8_ref[:], preferred_element_type=_f32, precision=jax.lax.Precision.HIGHEST)
    p1_ref[:] = (jnp.dot(xp, g1_ref[:], preferred_element_type=_f32, precision=jax.lax.Precision.HIGHEST)
                 + jnp.dot(a_s, g2_ref[:], preferred_element_type=_f32, precision=jax.lax.Precision.HIGHEST))
    d1_ref[:] = jnp.dot(a_d, p16_ref[:], preferred_element_type=_f32, precision=jax.lax.Precision.HIGHEST)
    init_ref[:] = (jnp.dot(xp * f_exp, g1_ref[:], preferred_element_type=_f32, precision=jax.lax.Precision.HIGHEST)
                   + jnp.dot(f_self, g2_ref[:], preferred_element_type=_f32, precision=jax.lax.Precision.HIGHEST))


def _mid_body(p0_ref, p1_ref, i1_ref, b1_ref, w2_ref, as2_ref, ad2_ref,
              hn_ref, hd_ref, k1_ref, k2_ref, k3_ref, one8_ref,
              t2_ref, d2_ref, init2_ref):
    s = p0_ref[:] + p1_ref[:] + i1_ref[:]
    num = jnp.dot(s, hn_ref[:], preferred_element_type=_f32, precision=jax.lax.Precision.HIGHEST)
    den = jnp.dot(s, hd_ref[:], preferred_element_type=_f32, precision=jax.lax.Precision.HIGHEST)
    h = jnp.maximum(num / den + b1_ref[:], 0.0)
    hp = jnp.dot(h, w2_ref[:], preferred_element_type=_f32)
    a2s = jnp.dot(hp, as2_ref[:], preferred_element_type=_f32, precision=jax.lax.Precision.HIGHEST)
    a2d = jnp.dot(hp, ad2_ref[:], preferred_element_type=_f32, precision=jax.lax.Precision.HIGHEST)
    al = a2s + a2d
    f2 = jnp.exp(jnp.maximum(al, 0.2 * al))
    f2e = jnp.dot(f2, one8_ref[:], preferred_element_type=_f32, precision=jax.lax.Precision.HIGHEST)
    t2_ref[:] = (jnp.dot(hp, k1_ref[:], preferred_element_type=_f32, precision=jax.lax.Precision.HIGHEST)
                 + jnp.dot(a2s, k2_ref[:], preferred_element_type=_f32, precision=jax.lax.Precision.HIGHEST))
    d2_ref[:] = jnp.dot(a2d, k3_ref[:], preferred_element_type=_f32, precision=jax.lax.Precision.HIGHEST)
    init2_ref[:] = (jnp.dot(hp * f2e, k1_ref[:], preferred_element_type=_f32, precision=jax.lax.Precision.HIGHEST)
                    + jnp.dot(f2, k2_ref[:], preferred_element_type=_f32, precision=jax.lax.Precision.HIGHEST))


def _fin_body(q0_ref, q1_ref, i2_ref, b2_ref, fw_ref, fb_ref, qn_ref, qd_ref,
              emb_ref, sc_ref):
    s = q0_ref[:] + q1_ref[:] + i2_ref[:]
    num = jnp.dot(s, qn_ref[:], preferred_element_type=_f32, precision=jax.lax.Precision.HIGHEST)
    den = jnp.dot(s, qd_ref[:], preferred_element_type=_f32, precision=jax.lax.Precision.HIGHEST)
    emb = num / den + b2_ref[:]
    emb_ref[:] = emb
    sc_ref[:] = jnp.dot(emb, fw_ref[:], preferred_element_type=_f32) + fb_ref[:]


def _make_edge_kernel(width):
    grp = 2 if width == _WL1 else 4
    ge = grp * _CH
    ng = _EPW // ge
    mesh = plsc.VectorSubcoreMesh(core_axis_name="c", subcore_axis_name="s",
                                  num_cores=_NC, num_subcores=_NS)
    nvec = width // 16

    def body(tbl, dtbl, eis, eid, out, acc,
             isrc0, idst0, rows0, drows0, isrc1, idst1, rows1, drows1,
             sem0, sem1, ssem0, ssem1):
        cid = lax.axis_index("c")
        sid = lax.axis_index("s")
        wid = sid * _NC + cid
        lane = lax.iota(jnp.int32, 16)
        zero16 = jnp.zeros((16,), _f32)
        bufs = [(isrc0, idst0, rows0, drows0, sem0),
                (isrc1, idst1, rows1, drows1, sem1)]

        def zrow(r, carry):
            for v in range(nvec):
                rows0[r, pl.ds(16 * v, 16)] = zero16
            return carry
        lax.fori_loop(0, ge, zrow, None)
        off = 0
        while off < _RPT:
            n = min(ge, _RPT - off)
            pltpu.sync_copy(rows0.at[pl.ds(0, n)],
                            acc.at[pl.ds(sid * _RPT + off, n)])
            off += n
        plsc.subcore_barrier()

        if width == _WL1:
            half = jnp.where(lane < 8, 0, 1).astype(jnp.int32)
            cidx = [64 + 2 * j + half for j in range(4)]

            def make_edge(rows_, drows_):
                def edge(e):
                    av = rows_[e, pl.ds(64, 16)]
                    dv = drows_[e, pl.ds(0, 16)]
                    al = av + dv
                    f = jnp.exp(jnp.maximum(al, 0.2 * al))
                    rows_[e, pl.ds(64, 16)] = f
                    es = jnp.full((16,), e, jnp.int32)
                    for j in range(4):
                        fe = plsc.load_gather(rows_, [es, cidx[j]])
                        rows_[e, pl.ds(16 * j, 16)] = fe * rows_[e, pl.ds(16 * j, 16)]
                return edge
        else:
            m8 = jnp.where(lane < 8, 1.0, 0.0).astype(_f32)
            e8 = jnp.where(lane == 8, 1.0, 0.0).astype(_f32)
            i8 = jnp.full((16,), 8, jnp.int32)
            i0 = jnp.zeros((16,), jnp.int32)

            def make_edge(rows_, drows_):
                def edge(e):
                    es = jnp.full((16,), e, jnp.int32)
                    av = rows_[e, pl.ds(0, 16)]
                    a2s = plsc.load_gather(rows_, [es, i8])
                    a2d = plsc.load_gather(drows_, [es, i0])
                    al = a2s + a2d
                    f = jnp.exp(jnp.maximum(al, 0.2 * al))
                    rows_[e, pl.ds(0, 16)] = f * (av * m8 + e8)
                return edge

        def fire(p, g):
            isrc_, idst_, rows_, drows_, sem_ = bufs[p]
            pltpu.sync_copy(eis.at[wid, pl.ds(g * grp, grp)], isrc_)
            pltpu.sync_copy(eid.at[wid, pl.ds(g * grp, grp)], idst_)
            for j in range(grp):
                pltpu.async_copy(tbl.at[isrc_.at[j]],
                                 rows_.at[pl.ds(j * _CH, _CH)], sem_)
                pltpu.async_copy(dtbl.at[idst_.at[j]],
                                 drows_.at[pl.ds(j * _CH, _CH)], sem_)

        def drain(p):
            isrc_, idst_, rows_, drows_, sem_ = bufs[p]
            for j in range(grp):
                pltpu.make_async_copy(tbl.at[isrc_.at[j]],
                                      rows_.at[pl.ds(j * _CH, _CH)], sem_).wait()
                pltpu.make_async_copy(dtbl.at[idst_.at[j]],
                                      drows_.at[pl.ds(j * _CH, _CH)], sem_).wait()

        def compute(p):
            _, _, rows_, drows_, _ = bufs[p]
            plsc.parallel_loop(0, ge, 1, unroll=16)(make_edge(rows_, drows_))

        def scat_fire(p, ssem_):
            _, idst_, rows_, _, _ = bufs[p]
            for j in range(grp):
                pltpu.async_copy(rows_.at[pl.ds(j * _CH, _CH)],
                                 acc.at[idst_.at[j]], ssem_, add=True)

        def scat_wait(p, ssem_):
            _, idst_, rows_, _, _ = bufs[p]
            for j in range(grp):
                pltpu.make_async_copy(rows_.at[pl.ds(j * _CH, _CH)],
                                      acc.at[idst_.at[j]], ssem_).wait()

        fire(0, 0)

        def pair(k, carry):
            g0 = 2 * k
            drain(0)
            fire(1, g0 + 1)
            compute(0)
            scat_fire(0, ssem0)
            drain(1)
            scat_wait(0, ssem0)
            @pl.when(k < ng // 2 - 1)
            def _():
                fire(0, g0 + 2)
            compute(1)
            scat_fire(1, ssem1)
            scat_wait(1, ssem1)
            return carry
        lax.fori_loop(0, ng // 2, pair, None)

        plsc.subcore_barrier()
        pltpu.sync_copy(acc.at[pl.ds(sid * _RPT, _RPT)],
                        out.at[cid, pl.ds(sid * _RPT, _RPT)])

    return pl.kernel(
        body,
        out_type=jax.ShapeDtypeStruct((_NC, _NPAD, width), _f32),
        mesh=mesh,
        compiler_params=pltpu.CompilerParams(use_tc_tiling_on_sc=False,
                                             needs_layout_passes=False),
        scratch_types=[
            pltpu.VMEM_SHARED((_NPAD, width), _f32),
            pltpu.VMEM((grp, _CH), jnp.int32),
            pltpu.VMEM((grp, _CH), jnp.int32),
            pltpu.VMEM((ge, width), _f32),
            pltpu.VMEM((ge, 16), _f32),
            pltpu.VMEM((grp, _CH), jnp.int32),
            pltpu.VMEM((grp, _CH), jnp.int32),
            pltpu.VMEM((ge, width), _f32),
            pltpu.VMEM((ge, 16), _f32),
            pltpu.SemaphoreType.DMA,
            pltpu.SemaphoreType.DMA,
            pltpu.SemaphoreType.DMA,
            pltpu.SemaphoreType.DMA,
        ],
    )


_edge_kernel = functools.cache(_make_edge_kernel)


def _static_mats():
    g1 = np.zeros((64, _WL1), np.float32)
    g1[:64, :64] = np.eye(64)
    g2 = np.zeros((8, _WL1), np.float32)
    g2[np.arange(8), 64 + np.arange(8)] = 1.0
    e8 = np.zeros((8, 64), np.float32)
    e8[np.arange(64) // 8, np.arange(64)] = 1.0
    p16 = np.zeros((8, 16), np.float32)
    p16[np.arange(8), np.arange(8)] = 1.0
    hn = np.zeros((_WL1, 64), np.float32)
    hn[:64, :64] = np.eye(64)
    hd = np.zeros((_WL1, 64), np.float32)
    hd[64 + np.arange(64) // 8, np.arange(64)] = 1.0
    k1 = np.zeros((8, _WL2), np.float32)
    k1[np.arange(8), np.arange(8)] = 1.0
    k2 = np.zeros((1, _WL2), np.float32)
    k2[0, 8] = 1.0
    k3 = np.zeros((1, _WL2), np.float32)
    k3[0, 0] = 1.0
    qn = np.zeros((_WL2, 8), np.float32)
    qn[:8, :8] = np.eye(8)
    qd = np.zeros((_WL2, 8), np.float32)
    qd[8, :] = 1.0
    one8 = np.ones((1, 8), np.float32)
    return dict(g1=g1, g2=g2, e8=e8, p16=p16, hn=hn, hd=hd,
                k1=k1, k2=k2, k3=k3, qn=qn, qd=qd, one8=one8)


_MATS = _static_mats()


def kernel(x, edge_index, W1, att_src1, att_dst1, b1, W2, att_src2, att_dst2,
           b2, fc3_w, fc3_b):
    m = _MATS
    ei = edge_index.astype(jnp.int32)
    npe = _EPAD - _E
    ar = jnp.arange(npe, dtype=jnp.int32)
    esrc = jnp.concatenate([ei[0], ar % _N])
    edst = jnp.concatenate([ei[1], _N + ar % (_NPAD - _N)])
    eis = esrc.reshape(_NW, _EPW // _CH, _CH)
    eid = edst.reshape(_NW, _EPW // _CH, _CH)

    ar64 = jnp.arange(64)
    ms = jnp.zeros((64, 8), _f32).at[ar64, ar64 // 8].set(att_src1.reshape(-1))
    md = jnp.zeros((64, 8), _f32).at[ar64, ar64 // 8].set(att_dst1.reshape(-1))

    blk = 2000
    full = lambda s: pl.BlockSpec(s, lambda i: (0, 0))
    rows = lambda c: pl.BlockSpec((blk, c), lambda i: (i, 0))
    p1t, d1t, init1 = pl.pallas_call(
        _prep1_body,
        grid=(_N // blk,),
        in_specs=[rows(128), full((128, 64)), full((64, 8)), full((64, 8)),
                  full((8, 64)), full((64, _WL1)), full((8, _WL1)),
                  full((8, 16))],
        out_specs=[rows(_WL1), rows(16), rows(_WL1)],
        out_shape=[jax.ShapeDtypeStruct((_N, _WL1), _f32),
                   jax.ShapeDtypeStruct((_N, 16), _f32),
                   jax.ShapeDtypeStruct((_N, _WL1), _f32)],
    )(x, W1, ms, md, m["e8"], m["g1"], m["g2"], m["p16"])

    d1p = jnp.zeros((_NPAD, 16), _f32).at[:_N].set(d1t)
    i1p = jnp.zeros((_NPAD, _WL1), _f32).at[:_N].set(init1)

    parts1 = _edge_kernel(_WL1)(p1t, d1p, eis, eid)

    blk2 = 1264
    rows2 = lambda c: pl.BlockSpec((blk2, c), lambda i: (i, 0))
    t2, d2, init2 = pl.pallas_call(
        _mid_body,
        grid=(_NPAD // blk2,),
        in_specs=[rows2(_WL1), rows2(_WL1), rows2(_WL1), full((1, 64)),
                  full((64, 8)), full((8, 1)), full((8, 1)),
                  full((_WL1, 64)), full((_WL1, 64)), full((8, _WL2)),
                  full((1, _WL2)), full((1, _WL2)), full((1, 8))],
        out_specs=[rows2(_WL2), rows2(_WL2), rows2(_WL2)],
        out_shape=[jax.ShapeDtypeStruct((_NPAD, _WL2), _f32),
                   jax.ShapeDtypeStruct((_NPAD, _WL2), _f32),
                   jax.ShapeDtypeStruct((_NPAD, _WL2), _f32)],
    )(parts1[0], parts1[1], i1p, b1.reshape(1, 64), W2,
      att_src2.reshape(8, 1), att_dst2.reshape(8, 1),
      m["hn"], m["hd"], m["k1"], m["k2"], m["k3"], m["one8"])

    parts2 = _edge_kernel(_WL2)(t2, d2, eis, eid)

    embp, scorep = pl.pallas_call(
        _fin_body,
        grid=(_NPAD // blk2,),
        in_specs=[rows2(_WL2), rows2(_WL2), rows2(_WL2), full((1, 8)),
                  full((8, 1)), full((1, 1)), full((_WL2, 8)),
                  full((_WL2, 8))],
        out_specs=[rows2(8), rows2(1)],
        out_shape=[jax.ShapeDtypeStruct((_NPAD, 8), _f32),
                   jax.ShapeDtypeStruct((_NPAD, 1), _f32)],
    )(parts2[0], parts2[1], init2, b2.reshape(1, 8), fc3_w,
      fc3_b.reshape(1, 1), m["qn"], m["qd"])

    return (embp[:_N], scorep[:_N])
